# baseline (speedup 1.0000x reference)
import functools

import numpy as np
import jax
import jax.numpy as jnp
from jax.experimental import pallas as pl
from jax.experimental.pallas import tpu as pltpu

D = 256
M = 256
ITER_N = 5
NHEAD = 5

_ROW_START = np.asarray([255 * i - (i * (i - 1)) // 2 for i in range(D)],
                        dtype=np.int32)
_TRIU_MASK = np.triu(np.ones((D, D), dtype=np.float32))


def _ns_sqrtm(A, eye):
    f32 = jnp.float32
    tr = jnp.sum(A * eye, keepdims=True)
    An = A * (1.0 / tr)
    I3 = 3.0 * eye
    ZY = 0.5 * (I3 - An)
    Y = jnp.dot(An, ZY, preferred_element_type=f32)
    Z = ZY
    for _ in range(ITER_N - 2):
        P = jnp.dot(Z, Y, preferred_element_type=f32)
        ZY = 0.5 * (I3 - P)
        YZ = jnp.concatenate([Y, Z], axis=0)
        YZn = jnp.dot(YZ, ZY, preferred_element_type=f32)
        Y = YZn[:D]
        Z = YZn[D:]
    P = jnp.dot(Z, Y, preferred_element_type=f32)
    YZY = 0.5 * jnp.dot(Y, I3 - P, preferred_element_type=f32)
    return YZY * jnp.sqrt(tr)


def _body(bb, feat_ref, wbn_ref, wscat_ref, out_ref):
    f32 = jnp.float32
    eye = (jax.lax.broadcasted_iota(jnp.int32, (D, D), 0)
           == jax.lax.broadcasted_iota(jnp.int32, (D, D), 1)).astype(f32)
    lane = jax.lax.broadcasted_iota(jnp.int32, (1, 128), 1)
    I3 = 3.0 * eye
    rng = range(bb)

    x = [jnp.dot(wbn_ref[...], feat_ref[b], preferred_element_type=f32)
         for b in rng]
    xc = [v - jnp.mean(v, axis=1, keepdims=True) for v in x]
    cov = [jax.lax.dot_general(v, v, (((1,), (1,)), ((), ())),
                               preferred_element_type=f32) * (1.0 / M)
           for v in xc]
    tr = [jnp.sum(A * eye, keepdims=True) for A in cov]
    An = [A * (1.0 / t) for A, t in zip(cov, tr)]
    ZY = [0.5 * (I3 - A) for A in An]
    Y = [jnp.dot(a, z, preferred_element_type=f32) for a, z in zip(An, ZY)]
    Z = ZY
    for _ in range(ITER_N - 2):
        P = [jnp.dot(z, y, preferred_element_type=f32) for z, y in zip(Z, Y)]
        ZY = [0.5 * (I3 - p) for p in P]
        YZn = [jnp.dot(jnp.concatenate([y, z], axis=0), zy,
                       preferred_element_type=f32)
               for y, z, zy in zip(Y, Z, ZY)]
        Y = [m[:D] for m in YZn]
        Z = [m[D:] for m in YZn]
    P = [jnp.dot(z, y, preferred_element_type=f32) for z, y in zip(Z, Y)]
    s = [0.5 * jnp.dot(y, I3 - p, preferred_element_type=f32)
         * jnp.sqrt(t)
         for y, p, t in zip(Y, P, tr)]
    vals = [[jnp.sum(sb * wscat_ref[k]) for k in range(NHEAD)] for sb in s]
    for b in rng:
        row = jnp.zeros((1, 128), f32)
        for k in range(NHEAD):
            row = jnp.where(lane == k, vals[b][k], row)
        out_ref[b] = row


@jax.jit
def kernel(features, w_bn, b_bn, w_type, b_type, w_flag, b_flag):
    B = features.shape[0]
    bb = 4
    feats = features.reshape(B, features.shape[1], M)

    w_all = jnp.concatenate([w_type, w_flag], axis=0)
    w_pad = jnp.pad(w_all, ((0, 0), (0, D)))
    rows = jnp.stack(
        [jax.lax.slice(w_pad, (0, int(t)), (NHEAD, int(t) + D))
         for t in _ROW_START],
        axis=1)
    w_scat = rows * _TRIU_MASK[None]

    out = pl.pallas_call(
        functools.partial(_body, bb),
        out_shape=jax.ShapeDtypeStruct((B, 1, 128), jnp.float32),
        grid=(B // bb,),
        in_specs=[
            pl.BlockSpec((bb, feats.shape[1], M), lambda i: (i, 0, 0)),
            pl.BlockSpec(w_bn.shape, lambda i: (0, 0)),
            pl.BlockSpec((NHEAD, D, D), lambda i: (0, 0, 0)),
        ],
        out_specs=pl.BlockSpec((bb, 1, 128), lambda i: (i, 0, 0)),
        compiler_params=pltpu.CompilerParams(
            dimension_semantics=("parallel",),
            vmem_limit_bytes=56 * 1024 * 1024,
        ),
        name="hpfnet_cov_pool",
    )(feats, w_bn, w_scat)

    logits = out[:, 0, :NHEAD]
    type_logits = logits[:, :4] + b_type
    flag_logits = logits[:, 4:5] + b_flag
    return (type_logits, flag_logits)


# speedup vs baseline: 1.0417x; 1.0417x over previous
import functools

import numpy as np
import jax
import jax.numpy as jnp
from jax.experimental import pallas as pl
from jax.experimental.pallas import tpu as pltpu

D = 256
M = 256
ITER_N = 5
NHEAD = 5

_ROW_START = np.asarray([255 * i - (i * (i - 1)) // 2 for i in range(D)],
                        dtype=np.int32)
_TRIU_MASK = np.triu(np.ones((D, D), dtype=np.float32))


def _ns_sqrtm(A, eye):
    f32 = jnp.float32
    tr = jnp.sum(A * eye, keepdims=True)
    An = A * (1.0 / tr)
    I3 = 3.0 * eye
    ZY = 0.5 * (I3 - An)
    Y = jnp.dot(An, ZY, preferred_element_type=f32)
    Z = ZY
    for _ in range(ITER_N - 2):
        P = jnp.dot(Z, Y, preferred_element_type=f32)
        ZY = 0.5 * (I3 - P)
        YZ = jnp.concatenate([Y, Z], axis=0)
        YZn = jnp.dot(YZ, ZY, preferred_element_type=f32)
        Y = YZn[:D]
        Z = YZn[D:]
    P = jnp.dot(Z, Y, preferred_element_type=f32)
    YZY = 0.5 * jnp.dot(Y, I3 - P, preferred_element_type=f32)
    return YZY * jnp.sqrt(tr)


def _body(bb, feat_ref, wbn_ref, wscat_ref, out_ref):
    f32 = jnp.float32
    eye = (jax.lax.broadcasted_iota(jnp.int32, (D, D), 0)
           == jax.lax.broadcasted_iota(jnp.int32, (D, D), 1)).astype(f32)
    lane = jax.lax.broadcasted_iota(jnp.int32, (1, 128), 1)
    I3 = 3.0 * eye
    rng = range(bb)

    x = [jnp.dot(wbn_ref[...], feat_ref[b], preferred_element_type=f32)
         for b in rng]
    xc = [v - jnp.mean(v, axis=1, keepdims=True) for v in x]
    cov = [jax.lax.dot_general(v, v, (((1,), (1,)), ((), ())),
                               preferred_element_type=f32) * (1.0 / M)
           for v in xc]
    tr = [jnp.sum(A * eye, keepdims=True) for A in cov]
    An = [A * (1.0 / t) for A, t in zip(cov, tr)]
    ZY = [0.5 * (I3 - A) for A in An]
    Y = [jnp.dot(a, z, preferred_element_type=f32) for a, z in zip(An, ZY)]
    Z = ZY
    for _ in range(ITER_N - 2):
        P = [jnp.dot(z, y, preferred_element_type=f32) for z, y in zip(Z, Y)]
        ZY = [0.5 * (I3 - p) for p in P]
        Yn = [jnp.dot(y, zy, preferred_element_type=f32)
              for y, zy in zip(Y, ZY)]
        Zn = [jnp.dot(zy, z, preferred_element_type=f32)
              for zy, z in zip(ZY, Z)]
        Y, Z = Yn, Zn
    P = [jnp.dot(z, y, preferred_element_type=f32) for z, y in zip(Z, Y)]
    s = [0.5 * jnp.dot(y, I3 - p, preferred_element_type=f32)
         * jnp.sqrt(t)
         for y, p, t in zip(Y, P, tr)]
    vals = [[jnp.sum(sb * wscat_ref[k]) for k in range(NHEAD)] for sb in s]
    for b in rng:
        row = jnp.zeros((1, 128), f32)
        for k in range(NHEAD):
            row = jnp.where(lane == k, vals[b][k], row)
        out_ref[b] = row


@jax.jit
def kernel(features, w_bn, b_bn, w_type, b_type, w_flag, b_flag):
    B = features.shape[0]
    bb = 8
    feats = features.reshape(B, features.shape[1], M)

    w_all = jnp.concatenate([w_type, w_flag], axis=0)
    w_pad = jnp.pad(w_all, ((0, 0), (0, D)))
    rows = jnp.stack(
        [jax.lax.slice(w_pad, (0, int(t)), (NHEAD, int(t) + D))
         for t in _ROW_START],
        axis=1)
    w_scat = rows * _TRIU_MASK[None]

    out = pl.pallas_call(
        functools.partial(_body, bb),
        out_shape=jax.ShapeDtypeStruct((B, 1, 128), jnp.float32),
        grid=(B // bb,),
        in_specs=[
            pl.BlockSpec((bb, feats.shape[1], M), lambda i: (i, 0, 0)),
            pl.BlockSpec(w_bn.shape, lambda i: (0, 0)),
            pl.BlockSpec((NHEAD, D, D), lambda i: (0, 0, 0)),
        ],
        out_specs=pl.BlockSpec((bb, 1, 128), lambda i: (i, 0, 0)),
        compiler_params=pltpu.CompilerParams(
            dimension_semantics=("parallel",),
            vmem_limit_bytes=56 * 1024 * 1024,
        ),
        name="hpfnet_cov_pool",
    )(feats, w_bn, w_scat)

    logits = out[:, 0, :NHEAD]
    type_logits = logits[:, :4] + b_type
    flag_logits = logits[:, 4:5] + b_flag
    return (type_logits, flag_logits)
